# baseline (device time: 170156 ns/iter reference)
import jax
import jax.numpy as jnp
from jax import lax
from jax.experimental import pallas as pl
from jax.experimental.pallas import tpu as pltpu

T_LOCAL = 1024
D = 1024
E_LOCAL = 8
E = 16
F = 4096

TF = 512
F_HALF_TILES = F // 2 // TF
CAP_H = 176
NC = 8
CHUNK = T_LOCAL // NC


def _router_exchange(r_shard):

    def body(r_ref, rg_ref, send_sem, recv_sem):
        my_x = lax.axis_index("x")
        my_y = lax.axis_index("y")

        barrier = pltpu.get_barrier_semaphore()
        pl.semaphore_signal(
            barrier, inc=1, device_id=(my_x, 1 - my_y),
            device_id_type=pl.DeviceIdType.MESH,
        )
        pl.semaphore_wait(barrier, 1)

        rg_ref[my_y] = r_ref[...]
        rdma = pltpu.make_async_remote_copy(
            src_ref=r_ref,
            dst_ref=rg_ref.at[my_y],
            send_sem=send_sem,
            recv_sem=recv_sem,
            device_id=(my_x, 1 - my_y),
            device_id_type=pl.DeviceIdType.MESH,
        )
        rdma.start()
        rdma.wait()

    return pl.pallas_call(
        body,
        out_shape=jax.ShapeDtypeStruct((2, D, E_LOCAL), jnp.float32),
        in_specs=[pl.BlockSpec(memory_space=pltpu.VMEM)],
        out_specs=pl.BlockSpec(memory_space=pltpu.VMEM),
        scratch_shapes=[pltpu.SemaphoreType.DMA, pltpu.SemaphoreType.DMA],
        compiler_params=pltpu.CompilerParams(collective_id=0),
    )(r_shard)


def _ffn_fused(scalars, xb, swl, wtl, swn, wtn, W1, W2):

    def tok_rdma(x_ref, xg_r, tsend, trecv, my_x, my_y):
        return pltpu.make_async_remote_copy(
            src_ref=x_ref,
            dst_ref=xg_r,
            send_sem=tsend,
            recv_sem=trecv,
            device_id=(my_x, 1 - my_y),
            device_id_type=pl.DeviceIdType.MESH,
        )

    def meta_rdma(src, dst, msend, mrecv, i, my_x, my_y):
        return pltpu.make_async_remote_copy(
            src_ref=src,
            dst_ref=dst,
            send_sem=msend.at[i],
            recv_sem=mrecv.at[i],
            device_id=(my_x, 1 - my_y),
            device_id_type=pl.DeviceIdType.MESH,
        )

    def body(xidx_ref, x_ref, swl_ref, wtl_ref, swn_ref, wtn_ref,
             w1_ref, w2_ref, out_ref,
             xg_r, swr_s, wtr_s, xd_s, gw_s, yacc_s,
             tsend, trecv, msend, mrecv):
        p0 = pl.program_id(0)
        e = pl.program_id(1)
        f = pl.program_id(2)
        my_x = lax.axis_index("x")
        my_y = lax.axis_index("y")

        @pl.when((p0 == 0) & (e == 0) & (f == 0))
        def _():
            barrier = pltpu.get_barrier_semaphore()
            pl.semaphore_signal(
                barrier, inc=1, device_id=(my_x, 1 - my_y),
                device_id_type=pl.DeviceIdType.MESH,
            )
            pl.semaphore_wait(barrier, 1)
            tok_rdma(x_ref, xg_r, tsend, trecv, my_x, my_y).start()
            meta_rdma(swn_ref, swr_s, msend, mrecv, 0, my_x, my_y).start()
            meta_rdma(wtn_ref, wtr_s, msend, mrecv, 1, my_x, my_y).start()

        @pl.when((p0 == 1) & (e == 0) & (f == 0))
        def _():
            tok_rdma(x_ref, xg_r, tsend, trecv, my_x, my_y).wait_recv()
            meta_rdma(swn_ref, swr_s, msend, mrecv, 0, my_x, my_y).wait_recv()
            meta_rdma(wtn_ref, wtr_s, msend, mrecv, 1, my_x, my_y).wait_recv()

        @pl.when(f == 0)
        def _():
            slots = lax.broadcasted_iota(jnp.int32, (CAP_H, T_LOCAL), 0)

            @pl.when(p0 == 0)
            def _():
                g = (slots == swl_ref[pl.ds(e, 1), :]).astype(jnp.bfloat16)
                gw_s[...] = g * wtl_ref[pl.ds(e, 1), :].astype(jnp.bfloat16)
                xd_s[...] = jnp.dot(
                    g, x_ref[...], preferred_element_type=jnp.float32
                ).astype(jnp.bfloat16)

            @pl.when(p0 == 1)
            def _():
                g = (slots == swr_s[pl.ds(e, 1), :]).astype(jnp.bfloat16)
                gw_s[...] = g * wtr_s[pl.ds(e, 1), :].astype(jnp.bfloat16)
                xd_s[...] = jnp.dot(
                    g, xg_r[...], preferred_element_type=jnp.float32
                ).astype(jnp.bfloat16)

        h1 = jnp.maximum(
            jnp.dot(xd_s[...], w1_ref[0].astype(jnp.bfloat16),
                    preferred_element_type=jnp.float32),
            0.0,
        )
        y = jnp.dot(
            h1.astype(jnp.bfloat16),
            w2_ref[0].astype(jnp.bfloat16),
            preferred_element_type=jnp.float32,
        )

        @pl.when(f == 0)
        def _():
            yacc_s[...] = y

        @pl.when(f > 0)
        def _():
            yacc_s[...] += y

        @pl.when(f == F_HALF_TILES - 1)
        def _():
            hh = jnp.where(p0 == 0, my_y, 1 - my_y)

            @pl.when((p0 == 0) & (e == 0))
            def _():
                out_ref[pl.ds(my_y * T_LOCAL, T_LOCAL), :] = (
                    jnp.zeros((T_LOCAL, D), jnp.float32)
                )

            @pl.when((p0 == 1) & (e == 0))
            def _():
                out_ref[pl.ds((1 - my_y) * T_LOCAL, T_LOCAL), :] = (
                    jnp.zeros((T_LOCAL, D), jnp.float32)
                )

            out_ref[pl.ds(hh * T_LOCAL, T_LOCAL), :] += lax.dot_general(
                gw_s[...], yacc_s[...].astype(jnp.bfloat16),
                (((0,), (0,)), ((), ())),
                preferred_element_type=jnp.float32,
            )

        @pl.when((p0 == 1) & (e == E_LOCAL - 1) & (f == F_HALF_TILES - 1))
        def _():
            tok_rdma(x_ref, xg_r, tsend, trecv, my_x, my_y).wait_send()
            meta_rdma(swn_ref, swr_s, msend, mrecv, 0, my_x, my_y).wait_send()
            meta_rdma(wtn_ref, wtr_s, msend, mrecv, 1, my_x, my_y).wait_send()

    grid = (2, E_LOCAL, F_HALF_TILES)
    grid_spec = pltpu.PrefetchScalarGridSpec(
        num_scalar_prefetch=1,
        grid=grid,
        in_specs=[
            pl.BlockSpec((T_LOCAL, D), lambda p, e, f, xi: (0, 0)),
            pl.BlockSpec((E_LOCAL, T_LOCAL), lambda p, e, f, xi: (0, 0)),
            pl.BlockSpec((E_LOCAL, T_LOCAL), lambda p, e, f, xi: (0, 0)),
            pl.BlockSpec((E_LOCAL, T_LOCAL), lambda p, e, f, xi: (0, 0)),
            pl.BlockSpec((E_LOCAL, T_LOCAL), lambda p, e, f, xi: (0, 0)),
            pl.BlockSpec(
                (1, D, TF),
                lambda p, e, f, xi: (e, 0, xi[0] * F_HALF_TILES + f),
            ),
            pl.BlockSpec(
                (1, TF, D),
                lambda p, e, f, xi: (e, xi[0] * F_HALF_TILES + f, 0),
            ),
        ],
        out_specs=pl.BlockSpec((2 * T_LOCAL, D), lambda p, e, f, xi: (0, 0)),
        scratch_shapes=[
            pltpu.VMEM((T_LOCAL, D), jnp.bfloat16),
            pltpu.VMEM((E_LOCAL, T_LOCAL), jnp.int32),
            pltpu.VMEM((E_LOCAL, T_LOCAL), jnp.float32),
            pltpu.VMEM((CAP_H, D), jnp.bfloat16),
            pltpu.VMEM((CAP_H, T_LOCAL), jnp.bfloat16),
            pltpu.VMEM((CAP_H, D), jnp.float32),
            pltpu.SemaphoreType.DMA,
            pltpu.SemaphoreType.DMA,
            pltpu.SemaphoreType.DMA((2,)),
            pltpu.SemaphoreType.DMA((2,)),
        ],
    )
    return pl.pallas_call(
        body,
        grid_spec=grid_spec,
        out_shape=jax.ShapeDtypeStruct((2 * T_LOCAL, D), jnp.float32),
        compiler_params=pltpu.CompilerParams(collective_id=1),
    )(scalars, xb, swl, wtl, swn, wtn, W1, W2)


def _combine(part_mine, part_theirs):

    def body(mine_ref, theirs_ref, out_ref, tb_s, ab_s, commy_ref,
             commx_ref, ysend, yrecv, xsend, xrecv):
        my_x = lax.axis_index("x")
        my_y = lax.axis_index("y")

        barrier = pltpu.get_barrier_semaphore()
        for dev in ((my_x, 1 - my_y), (1 - my_x, my_y)):
            pl.semaphore_signal(
                barrier, inc=1, device_id=dev,
                device_id_type=pl.DeviceIdType.MESH,
            )
        pl.semaphore_wait(barrier, 2)

        tb_s[...] = theirs_ref[...].astype(jnp.bfloat16)

        def rows(ref, c):
            return ref.at[pl.ds(c * CHUNK, CHUNK), :]

        y_rdmas = []
        for c in range(NC):
            r = pltpu.make_async_remote_copy(
                src_ref=rows(tb_s, c),
                dst_ref=rows(commy_ref, c),
                send_sem=ysend.at[c],
                recv_sem=yrecv.at[c],
                device_id=(my_x, 1 - my_y),
                device_id_type=pl.DeviceIdType.MESH,
            )
            r.start()
            y_rdmas.append(r)

        x_rdmas = []
        for c in range(NC):
            y_rdmas[c].wait_recv()
            sl = pl.ds(c * CHUNK, CHUNK)
            a = mine_ref[sl, :] + commy_ref[sl, :]
            out_ref[sl, :] = a
            ab_s[sl, :] = a.astype(jnp.bfloat16)
            r = pltpu.make_async_remote_copy(
                src_ref=rows(ab_s, c),
                dst_ref=rows(commx_ref, c),
                send_sem=xsend.at[c],
                recv_sem=xrecv.at[c],
                device_id=(1 - my_x, my_y),
                device_id_type=pl.DeviceIdType.MESH,
            )
            r.start()
            x_rdmas.append(r)

        for c in range(NC):
            x_rdmas[c].wait()
            sl = pl.ds(c * CHUNK, CHUNK)
            out_ref[sl, :] += commx_ref[sl, :]
        for c in range(NC):
            y_rdmas[c].wait_send()

    return pl.pallas_call(
        body,
        out_shape=jax.ShapeDtypeStruct((T_LOCAL, D), jnp.float32),
        in_specs=[
            pl.BlockSpec(memory_space=pltpu.VMEM),
            pl.BlockSpec(memory_space=pltpu.VMEM),
        ],
        out_specs=pl.BlockSpec(memory_space=pltpu.VMEM),
        scratch_shapes=[
            pltpu.VMEM((T_LOCAL, D), jnp.bfloat16),
            pltpu.VMEM((T_LOCAL, D), jnp.bfloat16),
            pltpu.VMEM((T_LOCAL, D), jnp.bfloat16),
            pltpu.VMEM((T_LOCAL, D), jnp.bfloat16),
            pltpu.SemaphoreType.DMA((NC,)),
            pltpu.SemaphoreType.DMA((NC,)),
            pltpu.SemaphoreType.DMA((NC,)),
            pltpu.SemaphoreType.DMA((NC,)),
        ],
        compiler_params=pltpu.CompilerParams(collective_id=2),
    )(part_mine, part_theirs)


def kernel(x, router, W1, W2):
    my_x = lax.axis_index("x")
    my_y = lax.axis_index("y")

    rg = _router_exchange(router)
    router_full = jnp.concatenate([rg[0], rg[1]], axis=1)
    gates = jnp.dot(x, router_full, precision=lax.Precision.HIGHEST)
    idx = jnp.arange(E)[None, :]
    m1 = jnp.max(gates, axis=1, keepdims=True)
    a1 = jnp.argmax(gates, axis=1)[:, None]
    masked = jnp.where(idx == a1, -jnp.inf, gates)
    m2 = jnp.max(masked, axis=1, keepdims=True)
    a2 = jnp.argmax(masked, axis=1)[:, None]
    b = jnp.exp(m2 - m1)
    denom = 1.0 + b
    w_dense = (
        jnp.where(idx == a1, 1.0 / denom, 0.0)
        + jnp.where(idx == a2, b / denom, 0.0)
    )
    assigned = w_dense > 0.0
    slot = jnp.cumsum(assigned.astype(jnp.int32), axis=0) - 1
    sw = jnp.where(assigned, slot, -1)

    def cols(full, grp):
        return lax.dynamic_slice(
            full, (0, grp * E_LOCAL), (T_LOCAL, E_LOCAL)
        ).T

    scalars = jnp.stack([my_x, my_y]).astype(jnp.int32)
    partial = _ffn_fused(
        scalars,
        x.astype(jnp.bfloat16),
        cols(sw, my_y), cols(w_dense, my_y),
        cols(sw, 1 - my_y), cols(w_dense, 1 - my_y),
        W1, W2,
    )

    mine = lax.dynamic_slice(partial, (my_y * T_LOCAL, 0), (T_LOCAL, D))
    theirs = lax.dynamic_slice(
        partial, ((1 - my_y) * T_LOCAL, 0), (T_LOCAL, D)
    )
    return _combine(mine, theirs)


# device time: 159413 ns/iter; 1.0674x vs baseline; 1.0674x over previous
import jax
import jax.numpy as jnp
from jax import lax
from jax.experimental import pallas as pl
from jax.experimental.pallas import tpu as pltpu

T_LOCAL = 1024
D = 1024
E_LOCAL = 8
E = 16
F = 4096

TF = 512
F_HALF_TILES = F // 2 // TF
CAP_H = 176
SLOTS = 2 * CAP_H
NC = 8
CHUNK = T_LOCAL // NC
NCX = 4
XCHUNK = T_LOCAL // 2 // NCX


def _exchange_route(x_shard, r_shard):

    def body(x_ref, r_ref, xg_ref, swg_ref, wtg_ref, swr_ref, wtr_ref,
             rg_s, rsend, rrecv, tsend, trecv, msend, mrecv):
        my_x = lax.axis_index("x")
        my_y = lax.axis_index("y")
        other = 1 - my_y

        barrier = pltpu.get_barrier_semaphore()
        pl.semaphore_signal(
            barrier, inc=1, device_id=(my_x, other),
            device_id_type=pl.DeviceIdType.MESH,
        )
        pl.semaphore_wait(barrier, 1)

        xg_ref[my_y] = x_ref[...].astype(jnp.bfloat16)
        tok = pltpu.make_async_remote_copy(
            src_ref=xg_ref.at[my_y],
            dst_ref=xg_ref.at[my_y],
            send_sem=tsend,
            recv_sem=trecv,
            device_id=(my_x, other),
            device_id_type=pl.DeviceIdType.MESH,
        )
        tok.start()

        rg_s[my_y] = r_ref[...]
        rr = pltpu.make_async_remote_copy(
            src_ref=r_ref,
            dst_ref=rg_s.at[my_y],
            send_sem=rsend,
            recv_sem=rrecv,
            device_id=(my_x, other),
            device_id_type=pl.DeviceIdType.MESH,
        )
        rr.start()
        rr.wait()

        rfull = jnp.concatenate([rg_s[0], rg_s[1]], axis=1)
        gates = jnp.dot(
            x_ref[...], rfull, preferred_element_type=jnp.float32,
            precision=lax.Precision.HIGHEST,
        )
        m1 = jnp.max(gates, axis=1, keepdims=True)
        t1 = gates == m1
        masked = jnp.where(t1, -jnp.inf, gates)
        m2 = jnp.max(masked, axis=1, keepdims=True)
        t2 = masked == m2
        bb = jnp.exp(m2 - m1)
        den = 1.0 + bb
        w = (
            t1.astype(jnp.float32) * (1.0 / den)
            + t2.astype(jnp.float32) * (bb / den)
        )
        assigned = t1 | t2
        ii = lax.broadcasted_iota(jnp.int32, (T_LOCAL, T_LOCAL), 0)
        jj = lax.broadcasted_iota(jnp.int32, (T_LOCAL, T_LOCAL), 1)
        tri = (ii >= jj).astype(jnp.bfloat16)
        cnt = jnp.dot(
            tri, assigned.astype(jnp.bfloat16),
            preferred_element_type=jnp.float32,
        )
        sw = jnp.where(assigned, cnt.astype(jnp.int32) - 1, -1)

        swg_ref[0] = sw[:, :E_LOCAL]
        swg_ref[1] = sw[:, E_LOCAL:]
        wtg_ref[0] = w[:, :E_LOCAL]
        wtg_ref[1] = w[:, E_LOCAL:]

        meta_rdmas = []
        for src, dst, i in ((swg_ref, swr_ref, 0), (wtg_ref, wtr_ref, 1)):
            r = pltpu.make_async_remote_copy(
                src_ref=src.at[other],
                dst_ref=dst,
                send_sem=msend.at[i],
                recv_sem=mrecv.at[i],
                device_id=(my_x, other),
                device_id_type=pl.DeviceIdType.MESH,
            )
            r.start()
            meta_rdmas.append(r)
        for r in meta_rdmas:
            r.wait()
        tok.wait()

    return pl.pallas_call(
        body,
        out_shape=(
            jax.ShapeDtypeStruct((2, T_LOCAL, D), jnp.bfloat16),
            jax.ShapeDtypeStruct((2, T_LOCAL, E_LOCAL), jnp.int32),
            jax.ShapeDtypeStruct((2, T_LOCAL, E_LOCAL), jnp.float32),
            jax.ShapeDtypeStruct((T_LOCAL, E_LOCAL), jnp.int32),
            jax.ShapeDtypeStruct((T_LOCAL, E_LOCAL), jnp.float32),
        ),
        in_specs=[
            pl.BlockSpec(memory_space=pltpu.VMEM),
            pl.BlockSpec(memory_space=pltpu.VMEM),
        ],
        out_specs=tuple(
            pl.BlockSpec(memory_space=pltpu.VMEM) for _ in range(5)
        ),
        scratch_shapes=[
            pltpu.VMEM((2, D, E_LOCAL), jnp.float32),
            pltpu.SemaphoreType.DMA,
            pltpu.SemaphoreType.DMA,
            pltpu.SemaphoreType.DMA,
            pltpu.SemaphoreType.DMA,
            pltpu.SemaphoreType.DMA((2,)),
            pltpu.SemaphoreType.DMA((2,)),
        ],
        compiler_params=pltpu.CompilerParams(collective_id=0),
    )(x_shard, r_shard)


def _ffn(my_x, xb, sw_t, wt_t, W1, W2):

    def body(xidx_ref, x_ref, sw_ref, wt_ref, w1_ref, w2_ref, out_ref,
             xd_s, gw_s, yacc_s):
        del xidx_ref
        e = pl.program_id(0)
        f = pl.program_id(1)

        @pl.when((e == 0) & (f == 0))
        def _():
            out_ref[...] = jnp.zeros_like(out_ref)

        @pl.when(f == 0)
        def _():
            slots = lax.broadcasted_iota(jnp.int32, (CAP_H, T_LOCAL), 0)
            sw = sw_ref[...]
            wtb = wt_ref[...].astype(jnp.bfloat16)
            for h in range(2):
                g = (slots == sw[0, h:h + 1, :]).astype(jnp.bfloat16)
                gw_s[pl.ds(h * CAP_H, CAP_H), :] = g * wtb[0, h:h + 1, :]
                xd_s[pl.ds(h * CAP_H, CAP_H), :] = jnp.dot(
                    g, x_ref[pl.ds(h * T_LOCAL, T_LOCAL), :],
                    preferred_element_type=jnp.float32,
                ).astype(jnp.bfloat16)

        h1 = jnp.maximum(
            jnp.dot(xd_s[...], w1_ref[0].astype(jnp.bfloat16),
                    preferred_element_type=jnp.float32),
            0.0,
        )
        y = jnp.dot(
            h1.astype(jnp.bfloat16),
            w2_ref[0].astype(jnp.bfloat16),
            preferred_element_type=jnp.float32,
        )

        @pl.when(f == 0)
        def _():
            yacc_s[...] = y

        @pl.when(f > 0)
        def _():
            yacc_s[...] += y

        @pl.when(f == F_HALF_TILES - 1)
        def _():
            yb = yacc_s[...].astype(jnp.bfloat16)
            for h in range(2):
                out_ref[pl.ds(h * T_LOCAL, T_LOCAL), :] += lax.dot_general(
                    gw_s[h * CAP_H:(h + 1) * CAP_H, :],
                    yb[h * CAP_H:(h + 1) * CAP_H, :],
                    (((0,), (0,)), ((), ())),
                    preferred_element_type=jnp.float32,
                )

    grid = (E_LOCAL, F_HALF_TILES)
    grid_spec = pltpu.PrefetchScalarGridSpec(
        num_scalar_prefetch=1,
        grid=grid,
        in_specs=[
            pl.BlockSpec((2 * T_LOCAL, D), lambda e, f, xi: (0, 0)),
            pl.BlockSpec((1, 2, T_LOCAL), lambda e, f, xi: (e, 0, 0)),
            pl.BlockSpec((1, 2, T_LOCAL), lambda e, f, xi: (e, 0, 0)),
            pl.BlockSpec(
                (1, D, TF),
                lambda e, f, xi: (e, 0, xi[0] * F_HALF_TILES + f),
            ),
            pl.BlockSpec(
                (1, TF, D),
                lambda e, f, xi: (e, xi[0] * F_HALF_TILES + f, 0),
            ),
        ],
        out_specs=pl.BlockSpec((2 * T_LOCAL, D), lambda e, f, xi: (0, 0)),
        scratch_shapes=[
            pltpu.VMEM((SLOTS, D), jnp.bfloat16),
            pltpu.VMEM((SLOTS, T_LOCAL), jnp.bfloat16),
            pltpu.VMEM((SLOTS, D), jnp.float32),
        ],
    )
    return pl.pallas_call(
        body,
        grid_spec=grid_spec,
        out_shape=jax.ShapeDtypeStruct((2 * T_LOCAL, D), jnp.float32),
    )(my_x.reshape(1), xb, sw_t, wt_t, W1, W2)


def _combine(part_mine, part_theirs):

    def body(mine_ref, theirs_ref, out_ref, tb_s, ab_s, commy_ref,
             commx_ref, ysend, yrecv, xsend, xrecv):
        my_x = lax.axis_index("x")
        my_y = lax.axis_index("y")

        barrier = pltpu.get_barrier_semaphore()
        for dev in ((my_x, 1 - my_y), (1 - my_x, my_y)):
            pl.semaphore_signal(
                barrier, inc=1, device_id=dev,
                device_id_type=pl.DeviceIdType.MESH,
            )
        pl.semaphore_wait(barrier, 2)

        tb_s[...] = theirs_ref[...].astype(jnp.bfloat16)

        def rows(ref, c):
            return ref.at[pl.ds(c * CHUNK, CHUNK), :]

        y_rdmas = []
        for c in range(NC):
            r = pltpu.make_async_remote_copy(
                src_ref=rows(tb_s, c),
                dst_ref=rows(commy_ref, c),
                send_sem=ysend.at[c],
                recv_sem=yrecv.at[c],
                device_id=(my_x, 1 - my_y),
                device_id_type=pl.DeviceIdType.MESH,
            )
            r.start()
            y_rdmas.append(r)

        x_rdmas = []
        for c in range(NC):
            y_rdmas[c].wait_recv()
            sl = pl.ds(c * CHUNK, CHUNK)
            a = mine_ref[sl, :] + commy_ref[sl, :]
            out_ref[sl, :] = a
            ab_s[sl, :] = a.astype(jnp.bfloat16)
            r = pltpu.make_async_remote_copy(
                src_ref=rows(ab_s, c),
                dst_ref=rows(commx_ref, c),
                send_sem=xsend.at[c],
                recv_sem=xrecv.at[c],
                device_id=(1 - my_x, my_y),
                device_id_type=pl.DeviceIdType.MESH,
            )
            r.start()
            x_rdmas.append(r)

        for c in range(NC):
            x_rdmas[c].wait()
            sl = pl.ds(c * CHUNK, CHUNK)
            out_ref[sl, :] += commx_ref[sl, :]
        for c in range(NC):
            y_rdmas[c].wait_send()

    return pl.pallas_call(
        body,
        out_shape=jax.ShapeDtypeStruct((T_LOCAL, D), jnp.float32),
        in_specs=[
            pl.BlockSpec(memory_space=pltpu.VMEM),
            pl.BlockSpec(memory_space=pltpu.VMEM),
        ],
        out_specs=pl.BlockSpec(memory_space=pltpu.VMEM),
        scratch_shapes=[
            pltpu.VMEM((T_LOCAL, D), jnp.bfloat16),
            pltpu.VMEM((T_LOCAL, D), jnp.bfloat16),
            pltpu.VMEM((T_LOCAL, D), jnp.bfloat16),
            pltpu.VMEM((T_LOCAL, D), jnp.bfloat16),
            pltpu.SemaphoreType.DMA((NC,)),
            pltpu.SemaphoreType.DMA((NC,)),
            pltpu.SemaphoreType.DMA((NC,)),
            pltpu.SemaphoreType.DMA((NC,)),
        ],
        compiler_params=pltpu.CompilerParams(collective_id=2),
    )(part_mine, part_theirs)


def kernel(x, router, W1, W2):
    my_x = lax.axis_index("x")
    my_y = lax.axis_index("y")

    xg, swg, wtg, swr, wtr = _exchange_route(x, router)
    xb = xg.reshape(2 * T_LOCAL, D)

    def asm(own, remote):
        mine = lax.dynamic_slice(
            own, (my_y, 0, 0), (1, T_LOCAL, E_LOCAL)
        )[0]
        h0 = jnp.where(my_y == 0, mine, remote)
        h1 = jnp.where(my_y == 0, remote, mine)
        return jnp.stack([h0.T, h1.T], axis=1)

    partial = _ffn(my_x, xb, asm(swg, swr), asm(wtg, wtr), W1, W2)

    mine = lax.dynamic_slice(partial, (my_y * T_LOCAL, 0), (T_LOCAL, D))
    theirs = lax.dynamic_slice(
        partial, ((1 - my_y) * T_LOCAL, 0), (T_LOCAL, D)
    )
    return _combine(mine, theirs)


# device time: 151597 ns/iter; 1.1224x vs baseline; 1.0516x over previous
import jax
import jax.numpy as jnp
from jax import lax
from jax.experimental import pallas as pl
from jax.experimental.pallas import tpu as pltpu

T_LOCAL = 1024
D = 1024
E_LOCAL = 8
E = 16
F = 4096

TF = 512
F_HALF_TILES = F // 2 // TF
CAP_H = 176
SLOTS = 2 * CAP_H
NC = 8
CHUNK = T_LOCAL // NC
NCX = 4
XCHUNK = T_LOCAL // 2 // NCX


def _exchange_route(x_shard, r_shard):

    def body(x_ref, r_ref, xg_ref, swg_ref, wtg_ref, swr_ref, wtr_ref,
             rg_s, rsend, rrecv, tsend, trecv, msend, mrecv):
        my_x = lax.axis_index("x")
        my_y = lax.axis_index("y")
        other = 1 - my_y

        barrier = pltpu.get_barrier_semaphore()
        pl.semaphore_signal(
            barrier, inc=1, device_id=(my_x, other),
            device_id_type=pl.DeviceIdType.MESH,
        )
        pl.semaphore_wait(barrier, 1)

        rg_s[my_y] = r_ref[...]
        rr = pltpu.make_async_remote_copy(
            src_ref=r_ref,
            dst_ref=rg_s.at[my_y],
            send_sem=rsend,
            recv_sem=rrecv,
            device_id=(my_x, other),
            device_id_type=pl.DeviceIdType.MESH,
        )
        rr.start()

        xg_ref[my_y] = x_ref[...].astype(jnp.bfloat16)
        tok = pltpu.make_async_remote_copy(
            src_ref=xg_ref.at[my_y],
            dst_ref=xg_ref.at[my_y],
            send_sem=tsend,
            recv_sem=trecv,
            device_id=(my_x, other),
            device_id_type=pl.DeviceIdType.MESH,
        )
        tok.start()
        rr.wait()

        rfull = jnp.concatenate([rg_s[0], rg_s[1]], axis=1)
        gates = jnp.dot(
            x_ref[...], rfull, preferred_element_type=jnp.float32,
            precision=lax.Precision.HIGHEST,
        )
        m1 = jnp.max(gates, axis=1, keepdims=True)
        t1 = gates == m1
        masked = jnp.where(t1, -jnp.inf, gates)
        m2 = jnp.max(masked, axis=1, keepdims=True)
        t2 = masked == m2
        bb = jnp.exp(m2 - m1)
        den = 1.0 + bb
        w = (
            t1.astype(jnp.float32) * (1.0 / den)
            + t2.astype(jnp.float32) * (bb / den)
        )
        assigned = t1 | t2
        ii = lax.broadcasted_iota(jnp.int32, (T_LOCAL, T_LOCAL), 0)
        jj = lax.broadcasted_iota(jnp.int32, (T_LOCAL, T_LOCAL), 1)
        tri = (ii >= jj).astype(jnp.bfloat16)
        cnt = jnp.dot(
            tri, assigned.astype(jnp.bfloat16),
            preferred_element_type=jnp.float32,
        )
        sw = jnp.where(assigned, cnt.astype(jnp.int32) - 1, -1)

        swg_ref[0] = sw[:, :E_LOCAL]
        swg_ref[1] = sw[:, E_LOCAL:]
        wtg_ref[0] = w[:, :E_LOCAL]
        wtg_ref[1] = w[:, E_LOCAL:]

        meta_rdmas = []
        for src, dst, i in ((swg_ref, swr_ref, 0), (wtg_ref, wtr_ref, 1)):
            r = pltpu.make_async_remote_copy(
                src_ref=src.at[other],
                dst_ref=dst,
                send_sem=msend.at[i],
                recv_sem=mrecv.at[i],
                device_id=(my_x, other),
                device_id_type=pl.DeviceIdType.MESH,
            )
            r.start()
            meta_rdmas.append(r)
        for r in meta_rdmas:
            r.wait()
        tok.wait()

    return pl.pallas_call(
        body,
        out_shape=(
            jax.ShapeDtypeStruct((2, T_LOCAL, D), jnp.bfloat16),
            jax.ShapeDtypeStruct((2, T_LOCAL, E_LOCAL), jnp.int32),
            jax.ShapeDtypeStruct((2, T_LOCAL, E_LOCAL), jnp.float32),
            jax.ShapeDtypeStruct((T_LOCAL, E_LOCAL), jnp.int32),
            jax.ShapeDtypeStruct((T_LOCAL, E_LOCAL), jnp.float32),
        ),
        in_specs=[
            pl.BlockSpec(memory_space=pltpu.VMEM),
            pl.BlockSpec(memory_space=pltpu.VMEM),
        ],
        out_specs=tuple(
            pl.BlockSpec(memory_space=pltpu.VMEM) for _ in range(5)
        ),
        scratch_shapes=[
            pltpu.VMEM((2, D, E_LOCAL), jnp.float32),
            pltpu.SemaphoreType.DMA,
            pltpu.SemaphoreType.DMA,
            pltpu.SemaphoreType.DMA,
            pltpu.SemaphoreType.DMA,
            pltpu.SemaphoreType.DMA((2,)),
            pltpu.SemaphoreType.DMA((2,)),
        ],
        compiler_params=pltpu.CompilerParams(collective_id=0),
    )(x_shard, r_shard)


def _ffn(my_x, xb, sw_t, wt_t, W1, W2):

    def body(xidx_ref, x_ref, sw_ref, wt_ref, w1_ref, w2_ref, out_ref,
             xd_s, gw_s, yacc_s):
        del xidx_ref
        e = pl.program_id(0)
        f = pl.program_id(1)

        @pl.when((e == 0) & (f == 0))
        def _():
            out_ref[...] = jnp.zeros_like(out_ref)

        @pl.when(f == 0)
        def _():
            slots = lax.broadcasted_iota(jnp.int32, (CAP_H, T_LOCAL), 0)
            sw = sw_ref[...]
            wtb = wt_ref[...].astype(jnp.bfloat16)
            for h in range(2):
                g = (slots == sw[0, h:h + 1, :]).astype(jnp.bfloat16)
                gw_s[pl.ds(h * CAP_H, CAP_H), :] = g * wtb[0, h:h + 1, :]
                xd_s[pl.ds(h * CAP_H, CAP_H), :] = jnp.dot(
                    g, x_ref[pl.ds(h * T_LOCAL, T_LOCAL), :],
                    preferred_element_type=jnp.float32,
                ).astype(jnp.bfloat16)

        h1 = jnp.maximum(
            jnp.dot(xd_s[...], w1_ref[0].astype(jnp.bfloat16),
                    preferred_element_type=jnp.float32),
            0.0,
        )
        y = jnp.dot(
            h1.astype(jnp.bfloat16),
            w2_ref[0].astype(jnp.bfloat16),
            preferred_element_type=jnp.float32,
        )

        @pl.when(f == 0)
        def _():
            yacc_s[...] = y

        @pl.when(f > 0)
        def _():
            yacc_s[...] += y

        @pl.when(f == F_HALF_TILES - 1)
        def _():
            yb = yacc_s[...].astype(jnp.bfloat16)
            for h in range(2):
                out_ref[pl.ds(h * T_LOCAL, T_LOCAL), :] += lax.dot_general(
                    gw_s[h * CAP_H:(h + 1) * CAP_H, :],
                    yb[h * CAP_H:(h + 1) * CAP_H, :],
                    (((0,), (0,)), ((), ())),
                    preferred_element_type=jnp.float32,
                )

    grid = (E_LOCAL, F_HALF_TILES)
    grid_spec = pltpu.PrefetchScalarGridSpec(
        num_scalar_prefetch=1,
        grid=grid,
        in_specs=[
            pl.BlockSpec((2 * T_LOCAL, D), lambda e, f, xi: (0, 0)),
            pl.BlockSpec((1, 2, T_LOCAL), lambda e, f, xi: (e, 0, 0)),
            pl.BlockSpec((1, 2, T_LOCAL), lambda e, f, xi: (e, 0, 0)),
            pl.BlockSpec(
                (1, D, TF),
                lambda e, f, xi: (e, 0, xi[0] * F_HALF_TILES + f),
            ),
            pl.BlockSpec(
                (1, TF, D),
                lambda e, f, xi: (e, xi[0] * F_HALF_TILES + f, 0),
            ),
        ],
        out_specs=pl.BlockSpec((2 * T_LOCAL, D), lambda e, f, xi: (0, 0)),
        scratch_shapes=[
            pltpu.VMEM((SLOTS, D), jnp.bfloat16),
            pltpu.VMEM((SLOTS, T_LOCAL), jnp.bfloat16),
            pltpu.VMEM((SLOTS, D), jnp.float32),
        ],
    )
    return pl.pallas_call(
        body,
        grid_spec=grid_spec,
        out_shape=jax.ShapeDtypeStruct((2 * T_LOCAL, D), jnp.float32),
    )(my_x.reshape(1), xb, sw_t, wt_t, W1, W2)


def _combine(part_mine, part_theirs):

    def body(mine_ref, theirs_ref, out_ref, tb_s, ab_s, commy_ref,
             commx_ref, ysend, yrecv, xsend, xrecv):
        my_x = lax.axis_index("x")
        my_y = lax.axis_index("y")

        barrier = pltpu.get_barrier_semaphore()
        for dev in ((my_x, 1 - my_y), (1 - my_x, my_y)):
            pl.semaphore_signal(
                barrier, inc=1, device_id=dev,
                device_id_type=pl.DeviceIdType.MESH,
            )
        pl.semaphore_wait(barrier, 2)

        tb_s[...] = theirs_ref[...].astype(jnp.bfloat16)

        def rows(ref, c):
            return ref.at[pl.ds(c * CHUNK, CHUNK), :]

        y_rdmas = []
        for c in range(NC):
            r = pltpu.make_async_remote_copy(
                src_ref=rows(tb_s, c),
                dst_ref=rows(commy_ref, c),
                send_sem=ysend.at[c],
                recv_sem=yrecv.at[c],
                device_id=(my_x, 1 - my_y),
                device_id_type=pl.DeviceIdType.MESH,
            )
            r.start()
            y_rdmas.append(r)

        x_rdmas = []
        for c in range(NC):
            y_rdmas[c].wait_recv()
            sl = pl.ds(c * CHUNK, CHUNK)
            a = mine_ref[sl, :] + commy_ref[sl, :]
            out_ref[sl, :] = a
            ab_s[sl, :] = a.astype(jnp.bfloat16)
            r = pltpu.make_async_remote_copy(
                src_ref=rows(ab_s, c),
                dst_ref=rows(commx_ref, c),
                send_sem=xsend.at[c],
                recv_sem=xrecv.at[c],
                device_id=(1 - my_x, my_y),
                device_id_type=pl.DeviceIdType.MESH,
            )
            r.start()
            x_rdmas.append(r)

        for c in range(NC):
            x_rdmas[c].wait()
            sl = pl.ds(c * CHUNK, CHUNK)
            out_ref[sl, :] += commx_ref[sl, :]
        for c in range(NC):
            y_rdmas[c].wait_send()

    return pl.pallas_call(
        body,
        out_shape=jax.ShapeDtypeStruct((T_LOCAL, D), jnp.float32),
        in_specs=[
            pl.BlockSpec(memory_space=pltpu.VMEM),
            pl.BlockSpec(memory_space=pltpu.VMEM),
        ],
        out_specs=pl.BlockSpec(memory_space=pltpu.VMEM),
        scratch_shapes=[
            pltpu.VMEM((T_LOCAL, D), jnp.bfloat16),
            pltpu.VMEM((T_LOCAL, D), jnp.bfloat16),
            pltpu.VMEM((T_LOCAL, D), jnp.bfloat16),
            pltpu.VMEM((T_LOCAL, D), jnp.bfloat16),
            pltpu.SemaphoreType.DMA((NC,)),
            pltpu.SemaphoreType.DMA((NC,)),
            pltpu.SemaphoreType.DMA((NC,)),
            pltpu.SemaphoreType.DMA((NC,)),
        ],
        compiler_params=pltpu.CompilerParams(collective_id=2),
    )(part_mine, part_theirs)


def kernel(x, router, W1, W2):
    my_x = lax.axis_index("x")
    my_y = lax.axis_index("y")

    xg, swg, wtg, swr, wtr = _exchange_route(x, router)
    xb = xg.reshape(2 * T_LOCAL, D)

    def asm(own, remote):
        mine = lax.dynamic_slice(
            own, (my_y, 0, 0), (1, T_LOCAL, E_LOCAL)
        )[0]
        h0 = jnp.where(my_y == 0, mine, remote)
        h1 = jnp.where(my_y == 0, remote, mine)
        return jnp.stack([h0.T, h1.T], axis=1)

    partial = _ffn(my_x, xb, asm(swg, swr), asm(wtg, wtr), W1, W2)

    mine = lax.dynamic_slice(partial, (my_y * T_LOCAL, 0), (T_LOCAL, D))
    theirs = lax.dynamic_slice(
        partial, ((1 - my_y) * T_LOCAL, 0), (T_LOCAL, D)
    )
    return _combine(mine, theirs)
